# Initial kernel scaffold; baseline (speedup 1.0000x reference)
import jax
import jax.numpy as jnp
from jax.experimental import pallas as pl


def kernel(x, table, W, b):
    raise NotImplementedError("write your pallas kernel here")


# speedup vs baseline: 1.2439x; 1.2439x over previous
import functools

import jax
import jax.numpy as jnp
from jax import lax
from jax.experimental import pallas as pl
from jax.experimental.pallas import tpu as pltpu
from jax.experimental.pallas import tpu_sc as plsc

NUM_EMBED = 1000000
EMBED_DIM = 64
OUTPUT_DIM = 128
BATCH = 16384
FIELDS = 26
ROWS = BATCH * FIELDS

NC = 2
NS = 16
NW = NC * NS
RPW = ROWS // NW
CH = 128
NCH = RPW // CH
GRP = CH // 16


def _gather_body(idx_hbm, table_hbm, h_hbm, idx_v, rows_v, gsem, ssem):
    wid = lax.axis_index("s") * NC + lax.axis_index("c")
    base = wid * RPW
    pltpu.sync_copy(idx_hbm.at[wid], idx_v)

    def fire_burst(c, slot):
        lanes = lax.iota(jnp.int32, 16)
        for g in range(GRP):
            vec = idx_v[pl.ds(c * CH + g * 16, 16)]
            for j in range(16):
                r = jnp.sum(vec * (lanes == j).astype(jnp.int32))
                pltpu.make_async_copy(
                    table_hbm.at[pl.ds(r, 1)],
                    rows_v.at[slot, pl.ds(g * 16 + j, 1)],
                    gsem.at[slot],
                ).start()

    def wait_burst(slot):
        pltpu.make_async_copy(
            table_hbm.at[pl.ds(0, CH)],
            rows_v.at[slot],
            gsem.at[slot],
        ).wait()

    def fire_store(c, slot):
        pltpu.make_async_copy(
            rows_v.at[slot],
            h_hbm.at[pl.ds(base + c * CH, CH)],
            ssem.at[slot],
        ).start()

    def wait_store(slot):
        pltpu.make_async_copy(
            rows_v.at[slot],
            h_hbm.at[pl.ds(base, CH)],
            ssem.at[slot],
        ).wait()

    fire_burst(0, 0)

    def step(c, carry):
        slot = c % 2

        @pl.when(c + 1 < NCH)
        def _():
            @pl.when(c >= 1)
            def _():
                wait_store(1 - slot)

            fire_burst(c + 1, 1 - slot)

        wait_burst(slot)
        fire_store(c, slot)
        return carry

    lax.fori_loop(0, NCH, step, 0)
    wait_store(0)
    wait_store(1)


@functools.cache
def _make_gather():
    return pl.kernel(
        _gather_body,
        mesh=plsc.VectorSubcoreMesh(core_axis_name="c", subcore_axis_name="s"),
        out_type=jax.ShapeDtypeStruct((ROWS, EMBED_DIM), jnp.float32),
        compiler_params=pltpu.CompilerParams(needs_layout_passes=False),
        scratch_types=[
            pltpu.VMEM((RPW,), jnp.int32),
            pltpu.VMEM((2, CH, EMBED_DIM), jnp.float32),
            pltpu.SemaphoreType.DMA((2,)),
            pltpu.SemaphoreType.DMA((2,)),
        ],
    )


MM_BLK = 2048


def _mm_body(h_ref, wt_ref, b_ref, o_ref):
    o_ref[...] = (
        jnp.dot(h_ref[...], wt_ref[...], preferred_element_type=jnp.float32)
        + b_ref[...]
    )


def _matmul(h, wt, b2d):
    return pl.pallas_call(
        _mm_body,
        grid=(ROWS // MM_BLK,),
        in_specs=[
            pl.BlockSpec((MM_BLK, EMBED_DIM), lambda i: (i, 0)),
            pl.BlockSpec((EMBED_DIM, OUTPUT_DIM), lambda i: (0, 0)),
            pl.BlockSpec((1, OUTPUT_DIM), lambda i: (0, 0)),
        ],
        out_specs=pl.BlockSpec((MM_BLK, OUTPUT_DIM), lambda i: (i, 0)),
        out_shape=jax.ShapeDtypeStruct((ROWS, OUTPUT_DIM), jnp.float32),
    )(h, wt, b2d)


def kernel(x, table, W, b):
    idx = x.T.reshape(NW, RPW).astype(jnp.int32)
    h = _make_gather()(idx, table)
    out = _matmul(h, W.T, b.reshape(1, OUTPUT_DIM))
    return out.reshape(FIELDS, BATCH, OUTPUT_DIM).transpose(1, 0, 2)
